# baseline (speedup 1.0000x reference)
import jax
import jax.numpy as jnp
from jax.experimental import pallas as pl

B, S, INPUT_LEN, D_MODEL, E = 4, 2048, 1024, 1024, 16
BLK = 4096


def _gating_kernel(x_ref, w1_ref, b1_ref, w2_ref, b2_ref, out_ref):
    x = x_ref[...]
    h = jnp.dot(x, w1_ref[...], preferred_element_type=jnp.float32)
    h = jnp.maximum(h + b1_ref[...], 0.0)
    lg = jnp.dot(h, w2_ref[...], preferred_element_type=jnp.float32)
    lg = lg + b2_ref[...]
    logits = lg.T

    idx = jax.lax.broadcasted_iota(jnp.int32, logits.shape, 0)
    m1 = jnp.max(logits, axis=0, keepdims=True)
    eq1 = logits >= m1
    i1 = jnp.min(jnp.where(eq1, idx, E), axis=0, keepdims=True)
    mask1 = idx == i1

    neg = jnp.float32(-jnp.inf)
    rest = jnp.where(mask1, neg, logits)
    m2 = jnp.max(rest, axis=0, keepdims=True)
    eq2 = rest >= m2
    i2 = jnp.min(jnp.where(eq2, idx, E), axis=0, keepdims=True)
    mask2 = idx == i2

    w_top = jax.nn.sigmoid(m1 - m2)
    res = jnp.where(mask1, w_top, 0.0) + jnp.where(mask2, 1.0 - w_top, 0.0)
    out_ref[...] = res.T


@jax.jit
def kernel(x, W1, b1, W2, b2):
    n_tok = B * S
    xf = x.reshape(n_tok, INPUT_LEN)
    b1r = b1.reshape(1, D_MODEL)
    b2r = b2.reshape(1, E)
    out = pl.pallas_call(
        _gating_kernel,
        grid=(n_tok // BLK,),
        in_specs=[
            pl.BlockSpec((BLK, INPUT_LEN), lambda i: (i, 0)),
            pl.BlockSpec((INPUT_LEN, D_MODEL), lambda i: (0, 0)),
            pl.BlockSpec((1, D_MODEL), lambda i: (0, 0)),
            pl.BlockSpec((D_MODEL, E), lambda i: (0, 0)),
            pl.BlockSpec((1, E), lambda i: (0, 0)),
        ],
        out_specs=pl.BlockSpec((BLK, E), lambda i: (i, 0)),
        out_shape=jax.ShapeDtypeStruct((n_tok, E), jnp.float32),
    )(xf, W1, b1r, W2, b2r)
    return out.reshape(B, S, E)


# speedup vs baseline: 1.0492x; 1.0492x over previous
import jax
import jax.numpy as jnp
from jax.experimental import pallas as pl

B, S, INPUT_LEN, D_MODEL, E = 4, 2048, 1024, 1024, 16
BLK = 2048


def _gating_kernel(x_ref, w1_ref, b1_ref, w2_ref, b2_ref, out_ref):
    x = x_ref[...]
    h = jnp.dot(x, w1_ref[...], preferred_element_type=jnp.float32)
    h = jnp.maximum(h + b1_ref[...], 0.0)
    lg = jnp.dot(h, w2_ref[...], preferred_element_type=jnp.float32)
    lg = lg + b2_ref[...]
    logits = lg.T

    idx = jax.lax.broadcasted_iota(jnp.int32, logits.shape, 0)
    m1 = jnp.max(logits, axis=0, keepdims=True)
    eq1 = logits >= m1
    i1 = jnp.min(jnp.where(eq1, idx, E), axis=0, keepdims=True)
    mask1 = idx == i1

    neg = jnp.float32(-jnp.inf)
    rest = jnp.where(mask1, neg, logits)
    m2 = jnp.max(rest, axis=0, keepdims=True)
    eq2 = rest >= m2
    i2 = jnp.min(jnp.where(eq2, idx, E), axis=0, keepdims=True)
    mask2 = idx == i2

    w_top = jax.nn.sigmoid(m1 - m2)
    res = jnp.where(mask1, w_top, 0.0) + jnp.where(mask2, 1.0 - w_top, 0.0)
    out_ref[...] = res.T


@jax.jit
def kernel(x, W1, b1, W2, b2):
    n_tok = B * S
    xf = x.reshape(n_tok, INPUT_LEN)
    b1r = b1.reshape(1, D_MODEL)
    b2r = b2.reshape(1, E)
    out = pl.pallas_call(
        _gating_kernel,
        grid=(n_tok // BLK,),
        in_specs=[
            pl.BlockSpec((BLK, INPUT_LEN), lambda i: (i, 0)),
            pl.BlockSpec((INPUT_LEN, D_MODEL), lambda i: (0, 0)),
            pl.BlockSpec((1, D_MODEL), lambda i: (0, 0)),
            pl.BlockSpec((D_MODEL, E), lambda i: (0, 0)),
            pl.BlockSpec((1, E), lambda i: (0, 0)),
        ],
        out_specs=pl.BlockSpec((BLK, E), lambda i: (i, 0)),
        out_shape=jax.ShapeDtypeStruct((n_tok, E), jnp.float32),
    )(xf, W1, b1r, W2, b2r)
    return out.reshape(B, S, E)


# speedup vs baseline: 1.0537x; 1.0043x over previous
import jax
import jax.numpy as jnp
from jax.experimental import pallas as pl
from jax.experimental.pallas import tpu as pltpu

B, S, INPUT_LEN, D_MODEL, E = 4, 2048, 1024, 1024, 16
BLK = 2048


def _gating_kernel(x_ref, w1_ref, b1_ref, w2_ref, b2_ref, out_ref):
    x = x_ref[...]
    h = jnp.dot(x, w1_ref[...], preferred_element_type=jnp.float32)
    h = jnp.maximum(h + b1_ref[...], 0.0)
    lg = jnp.dot(h, w2_ref[...], preferred_element_type=jnp.float32)
    lg = lg + b2_ref[...]
    logits = lg.T

    idx = jax.lax.broadcasted_iota(jnp.int32, logits.shape, 0)
    m1 = jnp.max(logits, axis=0, keepdims=True)
    eq1 = logits >= m1
    i1 = jnp.min(jnp.where(eq1, idx, E), axis=0, keepdims=True)
    mask1 = idx == i1

    neg = jnp.float32(-jnp.inf)
    rest = jnp.where(mask1, neg, logits)
    m2 = jnp.max(rest, axis=0, keepdims=True)
    eq2 = rest >= m2
    i2 = jnp.min(jnp.where(eq2, idx, E), axis=0, keepdims=True)
    mask2 = idx == i2

    w_top = jax.nn.sigmoid(m1 - m2)
    res = jnp.where(mask1, w_top, 0.0) + jnp.where(mask2, 1.0 - w_top, 0.0)
    out_ref[...] = res.T


@jax.jit
def kernel(x, W1, b1, W2, b2):
    n_tok = B * S
    xf = x.reshape(n_tok, INPUT_LEN)
    b1r = b1.reshape(1, D_MODEL)
    b2r = b2.reshape(1, E)
    out = pl.pallas_call(
        _gating_kernel,
        grid=(n_tok // BLK,),
        in_specs=[
            pl.BlockSpec((BLK, INPUT_LEN), lambda i: (i, 0)),
            pl.BlockSpec((INPUT_LEN, D_MODEL), lambda i: (0, 0)),
            pl.BlockSpec((1, D_MODEL), lambda i: (0, 0)),
            pl.BlockSpec((D_MODEL, E), lambda i: (0, 0)),
            pl.BlockSpec((1, E), lambda i: (0, 0)),
        ],
        out_specs=pl.BlockSpec((BLK, E), lambda i: (i, 0)),
        out_shape=jax.ShapeDtypeStruct((n_tok, E), jnp.float32),
        compiler_params=pltpu.CompilerParams(vmem_limit_bytes=100 * 1024 * 1024),
    )(xf, W1, b1r, W2, b2r)
    return out.reshape(B, S, E)
